# baseline (speedup 1.0000x reference)
import jax
import jax.numpy as jnp
from jax.experimental import pallas as pl
from jax.experimental.pallas import tpu as pltpu

_B = 4


def _sublane_suffix_max(y):
    neg = jnp.float32(-jnp.inf)
    d = 1
    while d < y.shape[1]:
        fill = jnp.full((y.shape[0], d, y.shape[2]), neg, y.dtype)
        y = jnp.maximum(y, jnp.concatenate([y[:, d:, :], fill], axis=1))
        d *= 2
    return y


def _corner_pool_kernel(x_ref, o_ref):
    y = _sublane_suffix_max(x_ref[...])
    y = jnp.swapaxes(y, 1, 2)
    y = _sublane_suffix_max(y)
    o_ref[...] = jnp.swapaxes(y + y, 1, 2)


@jax.jit
def kernel(x):
    N, C, H, W = x.shape
    xr = x.reshape(N * C, H, W)
    grid = (N * C // _B,)
    out = pl.pallas_call(
        _corner_pool_kernel,
        grid=grid,
        in_specs=[pl.BlockSpec((_B, H, W), lambda i: (i, 0, 0))],
        out_specs=pl.BlockSpec((_B, H, W), lambda i: (i, 0, 0)),
        out_shape=jax.ShapeDtypeStruct((N * C, H, W), x.dtype),
        compiler_params=pltpu.CompilerParams(
            dimension_semantics=("parallel",),
        ),
    )(xr)
    return out.reshape(N, C, H, W)


# speedup vs baseline: 2.1139x; 2.1139x over previous
import jax
import jax.numpy as jnp
from jax.experimental import pallas as pl
from jax.experimental.pallas import tpu as pltpu

_B = 16


def _sublane_suffix_max(y):
    neg = jnp.float32(-jnp.inf)
    d = 1
    while d < y.shape[1]:
        fill = jnp.full((y.shape[0], d, y.shape[2]), neg, y.dtype)
        y = jnp.maximum(y, jnp.concatenate([y[:, d:, :], fill], axis=1))
        d *= 2
    return y


def _corner_pool_kernel(x_ref, o_ref):
    y = _sublane_suffix_max(x_ref[...])
    y = jnp.swapaxes(y, 1, 2)
    y = _sublane_suffix_max(y)
    o_ref[...] = jnp.swapaxes(y + y, 1, 2)


@jax.jit
def kernel(x):
    N, C, H, W = x.shape
    xr = x.reshape(N * C, H, W)
    grid = (N * C // _B,)
    out = pl.pallas_call(
        _corner_pool_kernel,
        grid=grid,
        in_specs=[pl.BlockSpec((_B, H, W), lambda i: (i, 0, 0))],
        out_specs=pl.BlockSpec((_B, H, W), lambda i: (i, 0, 0)),
        out_shape=jax.ShapeDtypeStruct((N * C, H, W), x.dtype),
        compiler_params=pltpu.CompilerParams(
            dimension_semantics=("parallel",),
        ),
    )(xr)
    return out.reshape(N, C, H, W)


# speedup vs baseline: 2.6517x; 1.2544x over previous
import jax
import jax.numpy as jnp
from jax.experimental import pallas as pl
from jax.experimental.pallas import tpu as pltpu

_B = 32


def _sublane_suffix_max(y):
    neg = jnp.float32(-jnp.inf)
    d = 1
    while d < y.shape[1]:
        fill = jnp.full((y.shape[0], d, y.shape[2]), neg, y.dtype)
        y = jnp.maximum(y, jnp.concatenate([y[:, d:, :], fill], axis=1))
        d *= 2
    return y


def _corner_pool_kernel(x_ref, o_ref):
    y = _sublane_suffix_max(x_ref[...])
    y = jnp.swapaxes(y, 1, 2)
    y = _sublane_suffix_max(y)
    o_ref[...] = jnp.swapaxes(y + y, 1, 2)


@jax.jit
def kernel(x):
    N, C, H, W = x.shape
    xr = x.reshape(N * C, H, W)
    grid = (N * C // _B,)
    out = pl.pallas_call(
        _corner_pool_kernel,
        grid=grid,
        in_specs=[pl.BlockSpec((_B, H, W), lambda i: (i, 0, 0))],
        out_specs=pl.BlockSpec((_B, H, W), lambda i: (i, 0, 0)),
        out_shape=jax.ShapeDtypeStruct((N * C, H, W), x.dtype),
        compiler_params=pltpu.CompilerParams(
            dimension_semantics=("parallel",),
        ),
    )(xr)
    return out.reshape(N, C, H, W)


# speedup vs baseline: 3.0448x; 1.1483x over previous
import jax
import jax.numpy as jnp
from jax.experimental import pallas as pl
from jax.experimental.pallas import tpu as pltpu

_B = 64


def _sublane_suffix_max(y):
    neg = jnp.float32(-jnp.inf)
    d = 1
    while d < y.shape[1]:
        fill = jnp.full((y.shape[0], d, y.shape[2]), neg, y.dtype)
        y = jnp.maximum(y, jnp.concatenate([y[:, d:, :], fill], axis=1))
        d *= 2
    return y


def _corner_pool_kernel(x_ref, o_ref):
    y = _sublane_suffix_max(x_ref[...])
    y = jnp.swapaxes(y, 1, 2)
    y = _sublane_suffix_max(y)
    o_ref[...] = jnp.swapaxes(y + y, 1, 2)


@jax.jit
def kernel(x):
    N, C, H, W = x.shape
    xr = x.reshape(N * C, H, W)
    grid = (N * C // _B,)
    out = pl.pallas_call(
        _corner_pool_kernel,
        grid=grid,
        in_specs=[pl.BlockSpec((_B, H, W), lambda i: (i, 0, 0))],
        out_specs=pl.BlockSpec((_B, H, W), lambda i: (i, 0, 0)),
        out_shape=jax.ShapeDtypeStruct((N * C, H, W), x.dtype),
        compiler_params=pltpu.CompilerParams(
            dimension_semantics=("parallel",),
        ),
    )(xr)
    return out.reshape(N, C, H, W)


# speedup vs baseline: 3.1271x; 1.0270x over previous
import jax
import jax.numpy as jnp
from jax.experimental import pallas as pl
from jax.experimental.pallas import tpu as pltpu

_B = 128


def _sublane_suffix_max(y):
    neg = jnp.float32(-jnp.inf)
    d = 1
    while d < y.shape[1]:
        fill = jnp.full((y.shape[0], d, y.shape[2]), neg, y.dtype)
        y = jnp.maximum(y, jnp.concatenate([y[:, d:, :], fill], axis=1))
        d *= 2
    return y


def _corner_pool_kernel(x_ref, o_ref):
    y = _sublane_suffix_max(x_ref[...])
    y = jnp.swapaxes(y, 1, 2)
    y = _sublane_suffix_max(y)
    o_ref[...] = jnp.swapaxes(y + y, 1, 2)


@jax.jit
def kernel(x):
    N, C, H, W = x.shape
    xr = x.reshape(N * C, H, W)
    grid = (N * C // _B,)
    out = pl.pallas_call(
        _corner_pool_kernel,
        grid=grid,
        in_specs=[pl.BlockSpec((_B, H, W), lambda i: (i, 0, 0))],
        out_specs=pl.BlockSpec((_B, H, W), lambda i: (i, 0, 0)),
        out_shape=jax.ShapeDtypeStruct((N * C, H, W), x.dtype),
        compiler_params=pltpu.CompilerParams(
            dimension_semantics=("parallel",),
        ),
    )(xr)
    return out.reshape(N, C, H, W)
